# baseline (speedup 1.0000x reference)
import functools

import jax
import jax.numpy as jnp
from jax import lax
from jax.experimental import pallas as pl
from jax.experimental.pallas import tpu as pltpu
from jax.experimental.pallas import tpu_sc as plsc

B = 4
N = 4096
D_FEAT = 32
KNN = 16
TN = 512
NT = N // TN
P = B * N
NEG_INF = float("-inf")

HB = B // 2
P_SC = HB * N

NC = 2
NS = 16
NW = NC * NS
PTS_PER_W = P_SC // NW
CH = 128
NCHUNK = PTS_PER_W // CH
IDX_PER_CH = CH * KNN
GATHER_GRAIN = 128


def _topk_linear_kernel(pts_ref, ptsT_ref, f_ref, wab_ref, wb_ref, b1_ref,
                        idx_ref, a_ref, c_ref, fr_ref):
    b = pl.program_id(0)
    t = pl.program_id(1)

    p0 = pts_ref[0, 0:1, :]
    p1 = pts_ref[0, 1:2, :]
    p2 = pts_ref[0, 2:3, :]
    xx = p0 * p0 + p1 * p1 + p2 * p2

    q0 = ptsT_ref[0, :, 0:1]
    q1 = ptsT_ref[0, :, 1:2]
    q2 = ptsT_ref[0, :, 2:3]
    xxi = q0 * q0 + q1 * q1 + q2 * q2

    msum = jnp.dot(ptsT_ref[0], pts_ref[0],
                   preferred_element_type=jnp.float32)
    score = (-xxi + 2.0 * msum) - xx

    iota = lax.broadcasted_iota(jnp.int32, (TN, N), 1)
    u = lax.bitcast_convert_type(score, jnp.int32)
    inj = jnp.where(u < 0, iota & 0xFFF, ~iota & 0xFFF)
    packed = lax.bitcast_convert_type((u & ~0xFFF) | inj, jnp.float32)

    base = b * N
    m = jnp.max(packed, axis=1, keepdims=True)
    for s in range(KNN):
        m = jnp.max(jnp.where(packed < m, packed, NEG_INF),
                    axis=1, keepdims=True)
        mi = lax.bitcast_convert_type(m, jnp.int32)
        tail = mi & 0xFFF
        idx_ref[0, :, s:s + 1] = jnp.where(mi < 0, tail, 0xFFF - tail) + base

    ft = jnp.transpose(f_ref[0])
    fr_ref[0] = ft
    a_ref[0] = (jnp.dot(ft, wab_ref[...], preferred_element_type=jnp.float32)
                + b1_ref[0:1, :])
    c_ref[0] = jnp.dot(ft, wb_ref[...], preferred_element_type=jnp.float32)


def _run_topk_linear(pts, ptsT, feats, wabT, wbT, b1row):
    return pl.pallas_call(
        _topk_linear_kernel,
        grid=(HB, NT),
        in_specs=[
            pl.BlockSpec((1, 3, N), lambda b, t: (b, 0, 0)),
            pl.BlockSpec((1, TN, 3), lambda b, t: (b, t, 0)),
            pl.BlockSpec((1, D_FEAT, TN), lambda b, t: (b, 0, t)),
            pl.BlockSpec((D_FEAT, D_FEAT), lambda b, t: (0, 0)),
            pl.BlockSpec((D_FEAT, D_FEAT), lambda b, t: (0, 0)),
            pl.BlockSpec((1, D_FEAT), lambda b, t: (0, 0)),
        ],
        out_specs=[
            pl.BlockSpec((1, TN, KNN), lambda b, t: (b, t, 0)),
            pl.BlockSpec((1, TN, D_FEAT), lambda b, t: (b, t, 0)),
            pl.BlockSpec((1, TN, D_FEAT), lambda b, t: (b, t, 0)),
            pl.BlockSpec((1, TN, D_FEAT), lambda b, t: (b, t, 0)),
        ],
        out_shape=[
            jax.ShapeDtypeStruct((HB, N, KNN), jnp.int32),
            jax.ShapeDtypeStruct((HB, N, D_FEAT), jnp.float32),
            jax.ShapeDtypeStruct((HB, N, D_FEAT), jnp.float32),
            jax.ShapeDtypeStruct((HB, N, D_FEAT), jnp.float32),
        ],
    )(pts, ptsT, feats, wabT, wbT, b1row)


def _sc_body(crows_hbm, arows_hbm, frows_hbm, idx_hbm, out_hbm,
             idx_v, rows_v, a_v, f_v, o_v, sem):
    wid = lax.axis_index("s") * NC + lax.axis_index("c")
    base_pt = wid * PTS_PER_W

    def chunk_body(ci, carry):
        pt0 = pl.multiple_of(base_pt + ci * CH, CH)
        row0 = pl.multiple_of(pt0 * KNN // GATHER_GRAIN, IDX_PER_CH // GATHER_GRAIN)
        pltpu.sync_copy(idx_hbm.at[pl.ds(row0, IDX_PER_CH // GATHER_GRAIN)],
                        idx_v)
        copies = []
        for g in range(IDX_PER_CH // GATHER_GRAIN):
            copies.append(pltpu.async_copy(
                crows_hbm.at[idx_v.at[g]],
                rows_v.at[pl.ds(g * GATHER_GRAIN, GATHER_GRAIN)],
                sem))
        pltpu.sync_copy(arows_hbm.at[pl.ds(pt0, CH)], a_v)
        pltpu.sync_copy(frows_hbm.at[pl.ds(pt0, CH)], f_v)
        for cp in copies:
            cp.wait()

        def pt_body(p, c2):
            for h in (0, 16):
                a = a_v[p, pl.ds(h, 16)]
                acc = jnp.zeros((16,), jnp.float32)
                for j in range(KNN):
                    crow = rows_v[p * KNN + j, pl.ds(h, 16)]
                    acc = acc + jnp.maximum(a + crow, 0.0)
                o = jnp.maximum(f_v[p, pl.ds(h, 16)] + acc * (1.0 / KNN), 0.0)
                o_v[p, pl.ds(h, 16)] = o
            return c2

        lax.fori_loop(0, CH, pt_body, 0)
        pltpu.sync_copy(o_v, out_hbm.at[pl.ds(pt0, CH)])
        return carry

    lax.fori_loop(0, NCHUNK, chunk_body, 0)


def _run_sc(crows, arows, frows, idx2d):
    mesh = plsc.VectorSubcoreMesh(core_axis_name="c", subcore_axis_name="s")
    fn = functools.partial(
        pl.kernel, _sc_body, mesh=mesh,
        compiler_params=pltpu.CompilerParams(use_tc_tiling_on_sc=False),
        out_type=jax.ShapeDtypeStruct((P_SC, D_FEAT), jnp.float32),
        scratch_types=[
            pltpu.VMEM((IDX_PER_CH // GATHER_GRAIN, GATHER_GRAIN), jnp.int32),
            pltpu.VMEM((IDX_PER_CH, D_FEAT), jnp.float32),
            pltpu.VMEM((CH, D_FEAT), jnp.float32),
            pltpu.VMEM((CH, D_FEAT), jnp.float32),
            pltpu.VMEM((CH, D_FEAT), jnp.float32),
            pltpu.SemaphoreType.DMA,
        ],
    )()
    return fn(crows, arows, frows, idx2d)


def kernel(points, features, W1, b1):
    ptsT = jnp.transpose(points, (0, 2, 1))
    wa = W1[:, :D_FEAT]
    wb = W1[:, D_FEAT:]
    wabT = jnp.transpose(wa - wb)
    wbT = jnp.transpose(wb)
    b1row = b1.reshape(1, D_FEAT)

    halves = []
    for h in range(B // HB):
        sl = slice(h * HB, (h + 1) * HB)
        idx, arows3, crows3, frows3 = _run_topk_linear(
            points[sl], ptsT[sl], features[sl], wabT, wbT, b1row)
        idx2d = idx.reshape(P_SC * KNN // GATHER_GRAIN, GATHER_GRAIN)
        halves.append(_run_sc(crows3.reshape(P_SC, D_FEAT),
                              arows3.reshape(P_SC, D_FEAT),
                              frows3.reshape(P_SC, D_FEAT),
                              idx2d))
    out_rows = jnp.concatenate(halves, axis=0)
    return jnp.transpose(out_rows.reshape(B, N, D_FEAT), (0, 2, 1))


# speedup vs baseline: 1.0978x; 1.0978x over previous
import functools

import jax
import jax.numpy as jnp
from jax import lax
from jax.experimental import pallas as pl
from jax.experimental.pallas import tpu as pltpu
from jax.experimental.pallas import tpu_sc as plsc

B = 4
N = 4096
D_FEAT = 32
KNN = 16
TN = 512
NT = N // TN
P = B * N
NEG_INF = float("-inf")

HB = B // 2
P_SC = HB * N

NC = 2
NS = 16
NW = NC * NS
PTS_PER_W = P_SC // NW
CH = 128
NCHUNK = PTS_PER_W // CH
IDX_PER_CH = CH * KNN
GATHER_GRAIN = 128


def _topk_linear_kernel(pts_ref, ptsT_ref, f_ref, wab_ref, wb_ref, b1_ref,
                        idx_ref, a_ref, c_ref, fr_ref):
    b = pl.program_id(0)
    t = pl.program_id(1)

    p0 = pts_ref[0, 0:1, :]
    p1 = pts_ref[0, 1:2, :]
    p2 = pts_ref[0, 2:3, :]
    xx = p0 * p0 + p1 * p1 + p2 * p2

    q0 = ptsT_ref[0, :, 0:1]
    q1 = ptsT_ref[0, :, 1:2]
    q2 = ptsT_ref[0, :, 2:3]
    xxi = q0 * q0 + q1 * q1 + q2 * q2

    msum = jnp.dot(ptsT_ref[0], pts_ref[0],
                   preferred_element_type=jnp.float32)
    score = (-xxi + 2.0 * msum) - xx

    iota = lax.broadcasted_iota(jnp.int32, (TN, N), 1)
    u = lax.bitcast_convert_type(score, jnp.int32)
    inj = jnp.where(u < 0, iota & 0xFFF, ~iota & 0xFFF)
    packed = lax.bitcast_convert_type((u & ~0xFFF) | inj, jnp.float32)

    base = b * N
    hi = jnp.maximum(packed[:, :N // 2], packed[:, N // 2:])
    lo = jnp.minimum(packed[:, :N // 2], packed[:, N // 2:])
    m = jnp.max(hi, axis=1, keepdims=True)
    for s in range(KNN):
        cand = jnp.where(hi < m, hi, jnp.where(lo < m, lo, NEG_INF))
        m = jnp.max(cand, axis=1, keepdims=True)
        mi = lax.bitcast_convert_type(m, jnp.int32)
        tail = mi & 0xFFF
        idx_ref[0, :, s:s + 1] = jnp.where(mi < 0, tail, 0xFFF - tail) + base

    ft = jnp.transpose(f_ref[0])
    fr_ref[0] = ft
    a_ref[0] = (jnp.dot(ft, wab_ref[...], preferred_element_type=jnp.float32)
                + b1_ref[0:1, :])
    c_ref[0] = jnp.dot(ft, wb_ref[...], preferred_element_type=jnp.float32)


def _run_topk_linear(pts, ptsT, feats, wabT, wbT, b1row):
    return pl.pallas_call(
        _topk_linear_kernel,
        grid=(HB, NT),
        in_specs=[
            pl.BlockSpec((1, 3, N), lambda b, t: (b, 0, 0)),
            pl.BlockSpec((1, TN, 3), lambda b, t: (b, t, 0)),
            pl.BlockSpec((1, D_FEAT, TN), lambda b, t: (b, 0, t)),
            pl.BlockSpec((D_FEAT, D_FEAT), lambda b, t: (0, 0)),
            pl.BlockSpec((D_FEAT, D_FEAT), lambda b, t: (0, 0)),
            pl.BlockSpec((1, D_FEAT), lambda b, t: (0, 0)),
        ],
        out_specs=[
            pl.BlockSpec((1, TN, KNN), lambda b, t: (b, t, 0)),
            pl.BlockSpec((1, TN, D_FEAT), lambda b, t: (b, t, 0)),
            pl.BlockSpec((1, TN, D_FEAT), lambda b, t: (b, t, 0)),
            pl.BlockSpec((1, TN, D_FEAT), lambda b, t: (b, t, 0)),
        ],
        out_shape=[
            jax.ShapeDtypeStruct((HB, N, KNN), jnp.int32),
            jax.ShapeDtypeStruct((HB, N, D_FEAT), jnp.float32),
            jax.ShapeDtypeStruct((HB, N, D_FEAT), jnp.float32),
            jax.ShapeDtypeStruct((HB, N, D_FEAT), jnp.float32),
        ],
    )(pts, ptsT, feats, wabT, wbT, b1row)


def _sc_body(crows_hbm, arows_hbm, frows_hbm, idx_hbm, out_hbm,
             idx_v, rows_v, a_v, f_v, o_v, sem):
    wid = lax.axis_index("s") * NC + lax.axis_index("c")
    base_pt = wid * PTS_PER_W

    def chunk_body(ci, carry):
        pt0 = pl.multiple_of(base_pt + ci * CH, CH)
        row0 = pl.multiple_of(pt0 * KNN // GATHER_GRAIN, IDX_PER_CH // GATHER_GRAIN)
        pltpu.sync_copy(idx_hbm.at[pl.ds(row0, IDX_PER_CH // GATHER_GRAIN)],
                        idx_v)
        copies = []
        for g in range(IDX_PER_CH // GATHER_GRAIN):
            copies.append(pltpu.async_copy(
                crows_hbm.at[idx_v.at[g]],
                rows_v.at[pl.ds(g * GATHER_GRAIN, GATHER_GRAIN)],
                sem))
        pltpu.sync_copy(arows_hbm.at[pl.ds(pt0, CH)], a_v)
        pltpu.sync_copy(frows_hbm.at[pl.ds(pt0, CH)], f_v)
        for cp in copies:
            cp.wait()

        def pt_body(p, c2):
            for h in (0, 16):
                a = a_v[p, pl.ds(h, 16)]
                acc = jnp.zeros((16,), jnp.float32)
                for j in range(KNN):
                    crow = rows_v[p * KNN + j, pl.ds(h, 16)]
                    acc = acc + jnp.maximum(a + crow, 0.0)
                o = jnp.maximum(f_v[p, pl.ds(h, 16)] + acc * (1.0 / KNN), 0.0)
                o_v[p, pl.ds(h, 16)] = o
            return c2

        lax.fori_loop(0, CH, pt_body, 0)
        pltpu.sync_copy(o_v, out_hbm.at[pl.ds(pt0, CH)])
        return carry

    lax.fori_loop(0, NCHUNK, chunk_body, 0)


def _run_sc(crows, arows, frows, idx2d):
    mesh = plsc.VectorSubcoreMesh(core_axis_name="c", subcore_axis_name="s")
    fn = functools.partial(
        pl.kernel, _sc_body, mesh=mesh,
        compiler_params=pltpu.CompilerParams(use_tc_tiling_on_sc=False),
        out_type=jax.ShapeDtypeStruct((P_SC, D_FEAT), jnp.float32),
        scratch_types=[
            pltpu.VMEM((IDX_PER_CH // GATHER_GRAIN, GATHER_GRAIN), jnp.int32),
            pltpu.VMEM((IDX_PER_CH, D_FEAT), jnp.float32),
            pltpu.VMEM((CH, D_FEAT), jnp.float32),
            pltpu.VMEM((CH, D_FEAT), jnp.float32),
            pltpu.VMEM((CH, D_FEAT), jnp.float32),
            pltpu.SemaphoreType.DMA,
        ],
    )()
    return fn(crows, arows, frows, idx2d)


def kernel(points, features, W1, b1):
    ptsT = jnp.transpose(points, (0, 2, 1))
    wa = W1[:, :D_FEAT]
    wb = W1[:, D_FEAT:]
    wabT = jnp.transpose(wa - wb)
    wbT = jnp.transpose(wb)
    b1row = b1.reshape(1, D_FEAT)

    halves = []
    for h in range(B // HB):
        sl = slice(h * HB, (h + 1) * HB)
        idx, arows3, crows3, frows3 = _run_topk_linear(
            points[sl], ptsT[sl], features[sl], wabT, wbT, b1row)
        idx2d = idx.reshape(P_SC * KNN // GATHER_GRAIN, GATHER_GRAIN)
        halves.append(_run_sc(crows3.reshape(P_SC, D_FEAT),
                              arows3.reshape(P_SC, D_FEAT),
                              frows3.reshape(P_SC, D_FEAT),
                              idx2d))
    out_rows = jnp.concatenate(halves, axis=0)
    return jnp.transpose(out_rows.reshape(B, N, D_FEAT), (0, 2, 1))


# speedup vs baseline: 1.1275x; 1.0270x over previous
import functools

import jax
import jax.numpy as jnp
from jax import lax
from jax.experimental import pallas as pl
from jax.experimental.pallas import tpu as pltpu
from jax.experimental.pallas import tpu_sc as plsc

B = 4
N = 4096
D_FEAT = 32
KNN = 16
TN = 512
NT = N // TN
P = B * N
NEG_INF = float("-inf")

HB = B // 4
P_SC = HB * N

NC = 2
NS = 16
NW = NC * NS
PTS_PER_W = P_SC // NW
CH = 128
NCHUNK = PTS_PER_W // CH
IDX_PER_CH = CH * KNN
GATHER_GRAIN = 128


def _topk_linear_kernel(pts_ref, ptsT_ref, f_ref, wab_ref, wb_ref, b1_ref,
                        idx_ref, a_ref, c_ref, fr_ref):
    b = pl.program_id(0)
    t = pl.program_id(1)

    p0 = pts_ref[0, 0:1, :]
    p1 = pts_ref[0, 1:2, :]
    p2 = pts_ref[0, 2:3, :]
    xx = p0 * p0 + p1 * p1 + p2 * p2

    q0 = ptsT_ref[0, :, 0:1]
    q1 = ptsT_ref[0, :, 1:2]
    q2 = ptsT_ref[0, :, 2:3]
    xxi = q0 * q0 + q1 * q1 + q2 * q2

    msum = jnp.dot(ptsT_ref[0], pts_ref[0],
                   preferred_element_type=jnp.float32)
    score = (-xxi + 2.0 * msum) - xx

    iota = lax.broadcasted_iota(jnp.int32, (TN, N), 1)
    u = lax.bitcast_convert_type(score, jnp.int32)
    inj = jnp.where(u < 0, iota & 0xFFF, ~iota & 0xFFF)
    packed = lax.bitcast_convert_type((u & ~0xFFF) | inj, jnp.float32)

    base = b * N
    hi = jnp.maximum(packed[:, :N // 2], packed[:, N // 2:])
    lo = jnp.minimum(packed[:, :N // 2], packed[:, N // 2:])
    m = jnp.max(hi, axis=1, keepdims=True)
    for s in range(KNN):
        cand = jnp.where(hi < m, hi, jnp.where(lo < m, lo, NEG_INF))
        m = jnp.max(cand, axis=1, keepdims=True)
        mi = lax.bitcast_convert_type(m, jnp.int32)
        tail = mi & 0xFFF
        idx_ref[0, :, s:s + 1] = jnp.where(mi < 0, tail, 0xFFF - tail) + base

    ft = jnp.transpose(f_ref[0])
    fr_ref[0] = ft
    a_ref[0] = (jnp.dot(ft, wab_ref[...], preferred_element_type=jnp.float32)
                + b1_ref[0:1, :])
    c_ref[0] = jnp.dot(ft, wb_ref[...], preferred_element_type=jnp.float32)


def _run_topk_linear(pts, ptsT, feats, wabT, wbT, b1row):
    return pl.pallas_call(
        _topk_linear_kernel,
        grid=(HB, NT),
        in_specs=[
            pl.BlockSpec((1, 3, N), lambda b, t: (b, 0, 0)),
            pl.BlockSpec((1, TN, 3), lambda b, t: (b, t, 0)),
            pl.BlockSpec((1, D_FEAT, TN), lambda b, t: (b, 0, t)),
            pl.BlockSpec((D_FEAT, D_FEAT), lambda b, t: (0, 0)),
            pl.BlockSpec((D_FEAT, D_FEAT), lambda b, t: (0, 0)),
            pl.BlockSpec((1, D_FEAT), lambda b, t: (0, 0)),
        ],
        out_specs=[
            pl.BlockSpec((1, TN, KNN), lambda b, t: (b, t, 0)),
            pl.BlockSpec((1, TN, D_FEAT), lambda b, t: (b, t, 0)),
            pl.BlockSpec((1, TN, D_FEAT), lambda b, t: (b, t, 0)),
            pl.BlockSpec((1, TN, D_FEAT), lambda b, t: (b, t, 0)),
        ],
        out_shape=[
            jax.ShapeDtypeStruct((HB, N, KNN), jnp.int32),
            jax.ShapeDtypeStruct((HB, N, D_FEAT), jnp.float32),
            jax.ShapeDtypeStruct((HB, N, D_FEAT), jnp.float32),
            jax.ShapeDtypeStruct((HB, N, D_FEAT), jnp.float32),
        ],
    )(pts, ptsT, feats, wabT, wbT, b1row)


def _sc_body(crows_hbm, arows_hbm, frows_hbm, idx_hbm, out_hbm,
             idx_v, rows_v, a_v, f_v, o_v, sem):
    wid = lax.axis_index("s") * NC + lax.axis_index("c")
    base_pt = wid * PTS_PER_W

    def chunk_body(ci, carry):
        pt0 = pl.multiple_of(base_pt + ci * CH, CH)
        row0 = pl.multiple_of(pt0 * KNN // GATHER_GRAIN, IDX_PER_CH // GATHER_GRAIN)
        pltpu.sync_copy(idx_hbm.at[pl.ds(row0, IDX_PER_CH // GATHER_GRAIN)],
                        idx_v)
        copies = []
        for g in range(IDX_PER_CH // GATHER_GRAIN):
            copies.append(pltpu.async_copy(
                crows_hbm.at[idx_v.at[g]],
                rows_v.at[pl.ds(g * GATHER_GRAIN, GATHER_GRAIN)],
                sem))
        pltpu.sync_copy(arows_hbm.at[pl.ds(pt0, CH)], a_v)
        pltpu.sync_copy(frows_hbm.at[pl.ds(pt0, CH)], f_v)
        for cp in copies:
            cp.wait()

        def pt_body(p, c2):
            for h in (0, 16):
                a = a_v[p, pl.ds(h, 16)]
                acc = jnp.zeros((16,), jnp.float32)
                for j in range(KNN):
                    crow = rows_v[p * KNN + j, pl.ds(h, 16)]
                    acc = acc + jnp.maximum(a + crow, 0.0)
                o = jnp.maximum(f_v[p, pl.ds(h, 16)] + acc * (1.0 / KNN), 0.0)
                o_v[p, pl.ds(h, 16)] = o
            return c2

        lax.fori_loop(0, CH, pt_body, 0)
        pltpu.sync_copy(o_v, out_hbm.at[pl.ds(pt0, CH)])
        return carry

    lax.fori_loop(0, NCHUNK, chunk_body, 0)


def _run_sc(crows, arows, frows, idx2d):
    mesh = plsc.VectorSubcoreMesh(core_axis_name="c", subcore_axis_name="s")
    fn = functools.partial(
        pl.kernel, _sc_body, mesh=mesh,
        compiler_params=pltpu.CompilerParams(use_tc_tiling_on_sc=False),
        out_type=jax.ShapeDtypeStruct((P_SC, D_FEAT), jnp.float32),
        scratch_types=[
            pltpu.VMEM((IDX_PER_CH // GATHER_GRAIN, GATHER_GRAIN), jnp.int32),
            pltpu.VMEM((IDX_PER_CH, D_FEAT), jnp.float32),
            pltpu.VMEM((CH, D_FEAT), jnp.float32),
            pltpu.VMEM((CH, D_FEAT), jnp.float32),
            pltpu.VMEM((CH, D_FEAT), jnp.float32),
            pltpu.SemaphoreType.DMA,
        ],
    )()
    return fn(crows, arows, frows, idx2d)


def kernel(points, features, W1, b1):
    ptsT = jnp.transpose(points, (0, 2, 1))
    wa = W1[:, :D_FEAT]
    wb = W1[:, D_FEAT:]
    wabT = jnp.transpose(wa - wb)
    wbT = jnp.transpose(wb)
    b1row = b1.reshape(1, D_FEAT)

    halves = []
    for h in range(B // HB):
        sl = slice(h * HB, (h + 1) * HB)
        idx, arows3, crows3, frows3 = _run_topk_linear(
            points[sl], ptsT[sl], features[sl], wabT, wbT, b1row)
        idx2d = idx.reshape(P_SC * KNN // GATHER_GRAIN, GATHER_GRAIN)
        halves.append(_run_sc(crows3.reshape(P_SC, D_FEAT),
                              arows3.reshape(P_SC, D_FEAT),
                              frows3.reshape(P_SC, D_FEAT),
                              idx2d))
    out_rows = jnp.concatenate(halves, axis=0)
    return jnp.transpose(out_rows.reshape(B, N, D_FEAT), (0, 2, 1))


# speedup vs baseline: 1.1894x; 1.0549x over previous
import functools

import jax
import jax.numpy as jnp
from jax import lax
from jax.experimental import pallas as pl
from jax.experimental.pallas import tpu as pltpu
from jax.experimental.pallas import tpu_sc as plsc

B = 4
N = 4096
D_FEAT = 32
KNN = 16
TN = 512
NT = N // TN
P = B * N
NEG_INF = float("-inf")

HB = B // 4
P_SC = HB * N

NC = 2
NS = 16
NW = NC * NS
PTS_PER_W = P_SC // NW
CH = 128
NCHUNK = PTS_PER_W // CH
IDX_PER_CH = CH * KNN
GATHER_GRAIN = 128


def _topk_linear_kernel(pts_ref, ptsT_ref, f_ref, wab_ref, wb_ref, b1_ref,
                        idx_ref, a_ref, c_ref, fr_ref):
    b = pl.program_id(0)
    t = pl.program_id(1)

    p0 = pts_ref[0, 0:1, :]
    p1 = pts_ref[0, 1:2, :]
    p2 = pts_ref[0, 2:3, :]
    xx = p0 * p0 + p1 * p1 + p2 * p2

    q0 = ptsT_ref[0, :, 0:1]
    q1 = ptsT_ref[0, :, 1:2]
    q2 = ptsT_ref[0, :, 2:3]
    xxi = q0 * q0 + q1 * q1 + q2 * q2

    msum = jnp.dot(ptsT_ref[0], pts_ref[0],
                   preferred_element_type=jnp.float32)
    score = (-xxi + 2.0 * msum) - xx

    iota = lax.broadcasted_iota(jnp.int32, (TN, N), 1)
    u = lax.bitcast_convert_type(score, jnp.int32)
    inj = jnp.where(u < 0, iota & 0xFFF, ~iota & 0xFFF)
    packed = lax.bitcast_convert_type((u & ~0xFFF) | inj, jnp.float32)

    base = b * N
    qa = packed[:, :N // 4]
    qb = packed[:, N // 4:N // 2]
    qc = packed[:, N // 2:3 * N // 4]
    qd = packed[:, 3 * N // 4:]
    h1 = jnp.maximum(qa, qb)
    l1 = jnp.minimum(qa, qb)
    h2 = jnp.maximum(qc, qd)
    l2 = jnp.minimum(qc, qd)
    t1 = jnp.maximum(h1, h2)
    t4 = jnp.minimum(l1, l2)
    u1 = jnp.minimum(h1, h2)
    u2 = jnp.maximum(l1, l2)
    t2 = jnp.maximum(u1, u2)
    t3 = jnp.minimum(u1, u2)
    m = jnp.max(t1, axis=1, keepdims=True)
    for s in range(KNN):
        cand = jnp.where(t1 < m, t1,
                         jnp.where(t2 < m, t2,
                                   jnp.where(t3 < m, t3,
                                             jnp.where(t4 < m, t4, NEG_INF))))
        m = jnp.max(cand, axis=1, keepdims=True)
        mi = lax.bitcast_convert_type(m, jnp.int32)
        tail = mi & 0xFFF
        idx_ref[0, :, s:s + 1] = jnp.where(mi < 0, tail, 0xFFF - tail) + base

    ft = jnp.transpose(f_ref[0])
    fr_ref[0] = ft
    a_ref[0] = (jnp.dot(ft, wab_ref[...], preferred_element_type=jnp.float32)
                + b1_ref[0:1, :])
    c_ref[0] = jnp.dot(ft, wb_ref[...], preferred_element_type=jnp.float32)


def _run_topk_linear(pts, ptsT, feats, wabT, wbT, b1row):
    return pl.pallas_call(
        _topk_linear_kernel,
        grid=(HB, NT),
        in_specs=[
            pl.BlockSpec((1, 3, N), lambda b, t: (b, 0, 0)),
            pl.BlockSpec((1, TN, 3), lambda b, t: (b, t, 0)),
            pl.BlockSpec((1, D_FEAT, TN), lambda b, t: (b, 0, t)),
            pl.BlockSpec((D_FEAT, D_FEAT), lambda b, t: (0, 0)),
            pl.BlockSpec((D_FEAT, D_FEAT), lambda b, t: (0, 0)),
            pl.BlockSpec((1, D_FEAT), lambda b, t: (0, 0)),
        ],
        out_specs=[
            pl.BlockSpec((1, TN, KNN), lambda b, t: (b, t, 0)),
            pl.BlockSpec((1, TN, D_FEAT), lambda b, t: (b, t, 0)),
            pl.BlockSpec((1, TN, D_FEAT), lambda b, t: (b, t, 0)),
            pl.BlockSpec((1, TN, D_FEAT), lambda b, t: (b, t, 0)),
        ],
        out_shape=[
            jax.ShapeDtypeStruct((HB, N, KNN), jnp.int32),
            jax.ShapeDtypeStruct((HB, N, D_FEAT), jnp.float32),
            jax.ShapeDtypeStruct((HB, N, D_FEAT), jnp.float32),
            jax.ShapeDtypeStruct((HB, N, D_FEAT), jnp.float32),
        ],
    )(pts, ptsT, feats, wabT, wbT, b1row)


def _sc_body(crows_hbm, arows_hbm, frows_hbm, idx_hbm, out_hbm,
             idx_v, rows_v, a_v, f_v, o_v, sem):
    wid = lax.axis_index("s") * NC + lax.axis_index("c")
    base_pt = wid * PTS_PER_W

    def chunk_body(ci, carry):
        pt0 = pl.multiple_of(base_pt + ci * CH, CH)
        row0 = pl.multiple_of(pt0 * KNN // GATHER_GRAIN, IDX_PER_CH // GATHER_GRAIN)
        pltpu.sync_copy(idx_hbm.at[pl.ds(row0, IDX_PER_CH // GATHER_GRAIN)],
                        idx_v)
        copies = []
        for g in range(IDX_PER_CH // GATHER_GRAIN):
            copies.append(pltpu.async_copy(
                crows_hbm.at[idx_v.at[g]],
                rows_v.at[pl.ds(g * GATHER_GRAIN, GATHER_GRAIN)],
                sem))
        pltpu.sync_copy(arows_hbm.at[pl.ds(pt0, CH)], a_v)
        pltpu.sync_copy(frows_hbm.at[pl.ds(pt0, CH)], f_v)
        for cp in copies:
            cp.wait()

        def pt_body(p, c2):
            for h in (0, 16):
                a = a_v[p, pl.ds(h, 16)]
                acc = jnp.zeros((16,), jnp.float32)
                for j in range(KNN):
                    crow = rows_v[p * KNN + j, pl.ds(h, 16)]
                    acc = acc + jnp.maximum(a + crow, 0.0)
                o = jnp.maximum(f_v[p, pl.ds(h, 16)] + acc * (1.0 / KNN), 0.0)
                o_v[p, pl.ds(h, 16)] = o
            return c2

        lax.fori_loop(0, CH, pt_body, 0)
        pltpu.sync_copy(o_v, out_hbm.at[pl.ds(pt0, CH)])
        return carry

    lax.fori_loop(0, NCHUNK, chunk_body, 0)


def _run_sc(crows, arows, frows, idx2d):
    mesh = plsc.VectorSubcoreMesh(core_axis_name="c", subcore_axis_name="s")
    fn = functools.partial(
        pl.kernel, _sc_body, mesh=mesh,
        compiler_params=pltpu.CompilerParams(use_tc_tiling_on_sc=False),
        out_type=jax.ShapeDtypeStruct((P_SC, D_FEAT), jnp.float32),
        scratch_types=[
            pltpu.VMEM((IDX_PER_CH // GATHER_GRAIN, GATHER_GRAIN), jnp.int32),
            pltpu.VMEM((IDX_PER_CH, D_FEAT), jnp.float32),
            pltpu.VMEM((CH, D_FEAT), jnp.float32),
            pltpu.VMEM((CH, D_FEAT), jnp.float32),
            pltpu.VMEM((CH, D_FEAT), jnp.float32),
            pltpu.SemaphoreType.DMA,
        ],
    )()
    return fn(crows, arows, frows, idx2d)


def kernel(points, features, W1, b1):
    ptsT = jnp.transpose(points, (0, 2, 1))
    wa = W1[:, :D_FEAT]
    wb = W1[:, D_FEAT:]
    wabT = jnp.transpose(wa - wb)
    wbT = jnp.transpose(wb)
    b1row = b1.reshape(1, D_FEAT)

    halves = []
    for h in range(B // HB):
        sl = slice(h * HB, (h + 1) * HB)
        idx, arows3, crows3, frows3 = _run_topk_linear(
            points[sl], ptsT[sl], features[sl], wabT, wbT, b1row)
        idx2d = idx.reshape(P_SC * KNN // GATHER_GRAIN, GATHER_GRAIN)
        halves.append(_run_sc(crows3.reshape(P_SC, D_FEAT),
                              arows3.reshape(P_SC, D_FEAT),
                              frows3.reshape(P_SC, D_FEAT),
                              idx2d))
    out_rows = jnp.concatenate(halves, axis=0)
    return jnp.transpose(out_rows.reshape(B, N, D_FEAT), (0, 2, 1))
